# baseline (speedup 1.0000x reference)
import functools

import jax
import jax.numpy as jnp
from jax import lax
from jax.experimental import pallas as pl
from jax.experimental.pallas import tpu as pltpu
from jax.experimental.pallas import tpu_sc as plsc

N = 10000
E = 160000
DI = 256
DH = 256
DO = 128

E_PAD = 163840
EROWS = E_PAD // 128
NACC = 10240
DUMMY = 10008
MB = 10000
GRID_M = N // MB


def _deg_body(dst_hbm, out_hbm, dst_v, ones_v, zero_v, acc_sh):
    c = lax.axis_index("c")
    s = lax.axis_index("s")
    for i in range(8):
        ones_v[pl.ds(i * 16, 16)] = jnp.full((16,), 1.0, jnp.float32)

    def zfill(i, _):
        zero_v[pl.ds(i * 16, 16)] = jnp.zeros((16,), jnp.float32)
        return 0

    lax.fori_loop(0, 40, zfill, 0)
    pltpu.sync_copy(zero_v, acc_sh.at[pl.ds(pl.multiple_of(s * 640, 8), 640)])
    plsc.subcore_barrier()

    row0 = pl.multiple_of(c * 640 + s * 40, 8)
    pltpu.sync_copy(dst_hbm.at[pl.ds(row0, 40)], dst_v)

    def body(j, _):
        pltpu.sync_copy(ones_v, acc_sh.at[dst_v.at[j]], add=True)
        return 0

    lax.fori_loop(0, 40, body, 0)
    plsc.subcore_barrier()
    pltpu.sync_copy(acc_sh.at[pl.ds(pl.multiple_of(s * 640, 8), 640)],
                    out_hbm.at[pl.ds(pl.multiple_of(c * NACC + s * 640, 8), 640)])


@functools.cache
def _deg_call():
    mesh = plsc.VectorSubcoreMesh(core_axis_name="c", subcore_axis_name="s")
    return pl.kernel(
        _deg_body,
        mesh=mesh,
        out_type=jax.ShapeDtypeStruct((2 * NACC,), jnp.float32),
        scratch_types=[
            pltpu.VMEM((40, 128), jnp.int32),
            pltpu.VMEM((128,), jnp.float32),
            pltpu.VMEM((640,), jnp.float32),
            pltpu.VMEM_SHARED((NACC,), jnp.float32),
        ],
    )


def _agg_body(hs_hbm, src_hbm, dst_hbm, out_hbm, src_v, dst_v, buf0_v, buf1_v,
              acc_sh, gs0, gs1):
    c = lax.axis_index("c")
    s = lax.axis_index("s")
    r0 = pl.multiple_of(s * 624, 8)
    pltpu.sync_copy(hs_hbm.at[pl.ds(pl.multiple_of(c * N + r0, 8), 624)],
                    acc_sh.at[pl.ds(r0, 624)])

    @pl.when(s == 15)
    def _init_tail():
        pltpu.sync_copy(hs_hbm.at[pl.ds(pl.multiple_of(c * N + 9984, 8), 16)],
                        acc_sh.at[pl.ds(9984, 16)])

    off = c * N

    def load_idx(p):
        er = pl.multiple_of(s * 80 + p * 40, 8)
        pltpu.sync_copy(src_hbm.at[pl.ds(er, 40)], src_v)
        pltpu.sync_copy(dst_hbm.at[pl.ds(er, 40)], dst_v)

        def offs(r, _):
            for k in range(8):
                sl = pl.ds(k * 16, 16)
                src_v[r, sl] = src_v[r, sl] + off
            return 0

        lax.fori_loop(0, 40, offs, 0)

    load_idx(0)
    plsc.subcore_barrier()

    for p in range(2):
        if p:
            load_idx(p)
        pltpu.async_copy(hs_hbm.at[src_v.at[0]], buf0_v, gs0)

        def body(i, _):
            j0 = 2 * i
            j1 = 2 * i + 1
            pltpu.async_copy(hs_hbm.at[src_v.at[j1]], buf1_v, gs1)
            pltpu.make_async_copy(hs_hbm.at[src_v.at[j0]], buf0_v, gs0).wait()
            pltpu.sync_copy(buf0_v, acc_sh.at[dst_v.at[j0]], add=True)

            @pl.when(j1 + 1 < 40)
            def _prefetch():
                pltpu.async_copy(hs_hbm.at[src_v.at[j1 + 1]], buf0_v, gs0)

            pltpu.make_async_copy(hs_hbm.at[src_v.at[j1]], buf1_v, gs1).wait()
            pltpu.sync_copy(buf1_v, acc_sh.at[dst_v.at[j1]], add=True)
            return 0

        lax.fori_loop(0, 20, body, 0)
    plsc.subcore_barrier()
    pltpu.sync_copy(acc_sh.at[pl.ds(r0, 624)],
                    out_hbm.at[pl.ds(pl.multiple_of(c * N + r0, 8), 624)])

    @pl.when(s == 15)
    def _out_tail():
        pltpu.sync_copy(acc_sh.at[pl.ds(9984, 16)],
                        out_hbm.at[pl.ds(pl.multiple_of(c * N + 9984, 8), 16)])


@functools.cache
def _agg_call():
    mesh = plsc.VectorSubcoreMesh(core_axis_name="c", subcore_axis_name="s")
    return pl.kernel(
        _agg_body,
        mesh=mesh,
        out_type=jax.ShapeDtypeStruct((2 * N, DO), jnp.float32),
        scratch_types=[
            pltpu.VMEM((40, 128), jnp.int32),
            pltpu.VMEM((40, 128), jnp.int32),
            pltpu.VMEM((128, 128), jnp.float32),
            pltpu.VMEM((128, 128), jnp.float32),
            pltpu.VMEM_SHARED((NACC, 128), jnp.float32),
            pltpu.SemaphoreType.DMA,
            pltpu.SemaphoreType.DMA,
        ],
    )


def _mm1_body(dt_ref, x_ref, w_ref, out_ref):
    d = dt_ref[:, 0:1] + dt_ref[:, 1:2] + 1.0
    dinv = lax.rsqrt(d)
    h = jnp.dot(x_ref[...], w_ref[...], preferred_element_type=jnp.float32)
    out_ref[0, :, :] = h * dinv


def _mm1(deg_t, x, W1):
    return pl.pallas_call(
        _mm1_body,
        grid=(GRID_M, 2),
        in_specs=[
            pl.BlockSpec((MB, 2), lambda i, j: (i, 0)),
            pl.BlockSpec((MB, DI), lambda i, j: (i, 0)),
            pl.BlockSpec((DI, 128), lambda i, j: (0, j)),
        ],
        out_specs=pl.BlockSpec((1, MB, 128), lambda i, j: (j, i, 0)),
        out_shape=jax.ShapeDtypeStruct((2, N, 128), jnp.float32),
    )(deg_t, x, W1)


def _fin_body(dt_ref, a_ref, w2_ref, b1_ref, b2_ref, out_ref):
    d = dt_ref[:, 0:1] + dt_ref[:, 1:2] + 1.0
    dinv = lax.rsqrt(d)
    zA = a_ref[0] * dinv + b1_ref[0:1, 0:128]
    zB = a_ref[1] * dinv + b1_ref[0:1, 128:256]
    zA = jnp.where(zA > 0, zA, jnp.exp(jnp.minimum(zA, 0.0)) - 1.0)
    zB = jnp.where(zB > 0, zB, jnp.exp(jnp.minimum(zB, 0.0)) - 1.0)
    y = (jnp.dot(zA, w2_ref[0:128, :], preferred_element_type=jnp.float32)
         + jnp.dot(zB, w2_ref[128:256, :], preferred_element_type=jnp.float32)
         + b2_ref[0:1, :])
    y = jnp.maximum(y, 0.0)
    m = jnp.max(y, axis=1, keepdims=True)
    e = y - m
    out_ref[...] = e - jnp.log(jnp.sum(jnp.exp(e), axis=1, keepdims=True))


def _fin(deg_t, agg, W2, b1, b2):
    return pl.pallas_call(
        _fin_body,
        grid=(GRID_M,),
        in_specs=[
            pl.BlockSpec((MB, 2), lambda i: (i, 0)),
            pl.BlockSpec((2, MB, 128), lambda i: (0, i, 0)),
            pl.BlockSpec((DH, DO), lambda i: (0, 0)),
            pl.BlockSpec((1, DH), lambda i: (0, 0)),
            pl.BlockSpec((1, DO), lambda i: (0, 0)),
        ],
        out_specs=pl.BlockSpec((MB, DO), lambda i: (i, 0)),
        out_shape=jax.ShapeDtypeStruct((N, DO), jnp.float32),
    )(deg_t, agg, W2, b1, b2)


def kernel(x, edge_index, W1, b1, W2, b2):
    src = edge_index[0].astype(jnp.int32)
    dst = edge_index[1].astype(jnp.int32)
    pad = E_PAD - E
    src2 = jnp.concatenate([src, jnp.zeros((pad,), jnp.int32)]).reshape(EROWS, 128)
    dummy = DUMMY + (jnp.arange(pad, dtype=jnp.int32) % (NACC - DUMMY))
    dst2 = jnp.concatenate([dst, dummy]).reshape(EROWS, 128)

    deg2 = _deg_call()(dst2)
    deg_t = deg2.reshape(2, NACC).T
    hs3 = _mm1(deg_t, x, W1)
    agg = _agg_call()(hs3.reshape(2 * N, 128), src2, dst2)
    return _fin(deg_t, agg.reshape(2, N, 128), W2,
                b1.reshape(1, DH), b2.reshape(1, DO))


# speedup vs baseline: 1.0123x; 1.0123x over previous
import functools

import jax
import jax.numpy as jnp
from jax import lax
from jax.experimental import pallas as pl
from jax.experimental.pallas import tpu as pltpu
from jax.experimental.pallas import tpu_sc as plsc

N = 10000
E = 160000
DI = 256
DH = 256
DO = 128

E_PAD = 163840
EROWS = E_PAD // 128
NACC = 10240
DUMMY = 10008
MB = 5000
GRID_M = N // MB


def _deg_body(dst_hbm, out_hbm, dst_v, ones_v, zero_v, acc_sh):
    c = lax.axis_index("c")
    s = lax.axis_index("s")
    for i in range(8):
        ones_v[pl.ds(i * 16, 16)] = jnp.full((16,), 1.0, jnp.float32)

    def zfill(i, _):
        zero_v[pl.ds(i * 16, 16)] = jnp.zeros((16,), jnp.float32)
        return 0

    lax.fori_loop(0, 40, zfill, 0)
    pltpu.sync_copy(zero_v, acc_sh.at[pl.ds(pl.multiple_of(s * 640, 8), 640)])
    plsc.subcore_barrier()

    row0 = pl.multiple_of(c * 640 + s * 40, 8)
    pltpu.sync_copy(dst_hbm.at[pl.ds(row0, 40)], dst_v)

    def body(j, _):
        pltpu.sync_copy(ones_v, acc_sh.at[dst_v.at[j]], add=True)
        return 0

    lax.fori_loop(0, 40, body, 0)
    plsc.subcore_barrier()
    pltpu.sync_copy(acc_sh.at[pl.ds(pl.multiple_of(s * 640, 8), 640)],
                    out_hbm.at[pl.ds(pl.multiple_of(c * NACC + s * 640, 8), 640)])


@functools.cache
def _deg_call():
    mesh = plsc.VectorSubcoreMesh(core_axis_name="c", subcore_axis_name="s")
    return pl.kernel(
        _deg_body,
        mesh=mesh,
        out_type=jax.ShapeDtypeStruct((2 * NACC,), jnp.float32),
        scratch_types=[
            pltpu.VMEM((40, 128), jnp.int32),
            pltpu.VMEM((128,), jnp.float32),
            pltpu.VMEM((640,), jnp.float32),
            pltpu.VMEM_SHARED((NACC,), jnp.float32),
        ],
    )


def _agg_body(hs_hbm, src_hbm, dst_hbm, out_hbm, src_v, dst_v, buf0_v, buf1_v,
              acc_sh, gs0, gs1, isem):
    c = lax.axis_index("c")
    s = lax.axis_index("s")
    r0 = pl.multiple_of(s * 624, 8)
    pltpu.async_copy(hs_hbm.at[pl.ds(pl.multiple_of(c * N + r0, 8), 624)],
                     acc_sh.at[pl.ds(r0, 624)], isem)

    @pl.when(s == 15)
    def _init_tail():
        pltpu.sync_copy(hs_hbm.at[pl.ds(pl.multiple_of(c * N + 9984, 8), 16)],
                        acc_sh.at[pl.ds(9984, 16)])

    off = c * N

    def load_idx(p):
        er = pl.multiple_of(s * 80 + p * 40, 8)
        pltpu.sync_copy(src_hbm.at[pl.ds(er, 40)], src_v)
        pltpu.sync_copy(dst_hbm.at[pl.ds(er, 40)], dst_v)

        def offs(r, _):
            for k in range(8):
                sl = pl.ds(k * 16, 16)
                src_v[r, sl] = src_v[r, sl] + off
            return 0

        lax.fori_loop(0, 40, offs, 0)

    load_idx(0)
    pltpu.async_copy(hs_hbm.at[src_v.at[0]], buf0_v, gs0)
    pltpu.make_async_copy(hs_hbm.at[pl.ds(pl.multiple_of(c * N + r0, 8), 624)],
                          acc_sh.at[pl.ds(r0, 624)], isem).wait()
    plsc.subcore_barrier()

    for p in range(2):
        if p:
            load_idx(p)
            pltpu.async_copy(hs_hbm.at[src_v.at[0]], buf0_v, gs0)

        def body(i, _):
            j0 = 2 * i
            j1 = 2 * i + 1
            pltpu.async_copy(hs_hbm.at[src_v.at[j1]], buf1_v, gs1)
            pltpu.make_async_copy(hs_hbm.at[src_v.at[j0]], buf0_v, gs0).wait()
            pltpu.sync_copy(buf0_v, acc_sh.at[dst_v.at[j0]], add=True)

            @pl.when(j1 + 1 < 40)
            def _prefetch():
                pltpu.async_copy(hs_hbm.at[src_v.at[j1 + 1]], buf0_v, gs0)

            pltpu.make_async_copy(hs_hbm.at[src_v.at[j1]], buf1_v, gs1).wait()
            pltpu.sync_copy(buf1_v, acc_sh.at[dst_v.at[j1]], add=True)
            return 0

        lax.fori_loop(0, 20, body, 0)
    plsc.subcore_barrier()
    pltpu.sync_copy(acc_sh.at[pl.ds(r0, 624)],
                    out_hbm.at[pl.ds(pl.multiple_of(c * N + r0, 8), 624)])

    @pl.when(s == 15)
    def _out_tail():
        pltpu.sync_copy(acc_sh.at[pl.ds(9984, 16)],
                        out_hbm.at[pl.ds(pl.multiple_of(c * N + 9984, 8), 16)])


@functools.cache
def _agg_call():
    mesh = plsc.VectorSubcoreMesh(core_axis_name="c", subcore_axis_name="s")
    return pl.kernel(
        _agg_body,
        mesh=mesh,
        out_type=jax.ShapeDtypeStruct((2 * N, DO), jnp.float32),
        scratch_types=[
            pltpu.VMEM((40, 128), jnp.int32),
            pltpu.VMEM((40, 128), jnp.int32),
            pltpu.VMEM((128, 128), jnp.float32),
            pltpu.VMEM((128, 128), jnp.float32),
            pltpu.VMEM_SHARED((NACC, 128), jnp.float32),
            pltpu.SemaphoreType.DMA,
            pltpu.SemaphoreType.DMA,
            pltpu.SemaphoreType.DMA,
        ],
    )


def _mm1_body(dt_ref, x_ref, w_ref, out_ref):
    d = dt_ref[:, 0:1] + dt_ref[:, 1:2] + 1.0
    dinv = lax.rsqrt(d)
    h = jnp.dot(x_ref[...], w_ref[...], preferred_element_type=jnp.float32)
    out_ref[0, :, :] = h * dinv


def _mm1(deg_t, x, W1):
    return pl.pallas_call(
        _mm1_body,
        grid=(GRID_M, 2),
        in_specs=[
            pl.BlockSpec((MB, 2), lambda i, j: (i, 0)),
            pl.BlockSpec((MB, DI), lambda i, j: (i, 0)),
            pl.BlockSpec((DI, 128), lambda i, j: (0, j)),
        ],
        out_specs=pl.BlockSpec((1, MB, 128), lambda i, j: (j, i, 0)),
        out_shape=jax.ShapeDtypeStruct((2, N, 128), jnp.float32),
    )(deg_t, x, W1)


def _fin_body(dt_ref, a_ref, w2_ref, b1_ref, b2_ref, out_ref):
    d = dt_ref[:, 0:1] + dt_ref[:, 1:2] + 1.0
    dinv = lax.rsqrt(d)
    zA = a_ref[0] * dinv + b1_ref[0:1, 0:128]
    zB = a_ref[1] * dinv + b1_ref[0:1, 128:256]
    zA = jnp.where(zA > 0, zA, jnp.exp(jnp.minimum(zA, 0.0)) - 1.0)
    zB = jnp.where(zB > 0, zB, jnp.exp(jnp.minimum(zB, 0.0)) - 1.0)
    y = (jnp.dot(zA, w2_ref[0:128, :], preferred_element_type=jnp.float32)
         + jnp.dot(zB, w2_ref[128:256, :], preferred_element_type=jnp.float32)
         + b2_ref[0:1, :])
    y = jnp.maximum(y, 0.0)
    m = jnp.max(y, axis=1, keepdims=True)
    e = y - m
    out_ref[...] = e - jnp.log(jnp.sum(jnp.exp(e), axis=1, keepdims=True))


def _fin(deg_t, agg, W2, b1, b2):
    return pl.pallas_call(
        _fin_body,
        grid=(GRID_M,),
        in_specs=[
            pl.BlockSpec((MB, 2), lambda i: (i, 0)),
            pl.BlockSpec((2, MB, 128), lambda i: (0, i, 0)),
            pl.BlockSpec((DH, DO), lambda i: (0, 0)),
            pl.BlockSpec((1, DH), lambda i: (0, 0)),
            pl.BlockSpec((1, DO), lambda i: (0, 0)),
        ],
        out_specs=pl.BlockSpec((MB, DO), lambda i: (i, 0)),
        out_shape=jax.ShapeDtypeStruct((N, DO), jnp.float32),
    )(deg_t, agg, W2, b1, b2)


def kernel(x, edge_index, W1, b1, W2, b2):
    src = edge_index[0].astype(jnp.int32)
    dst = edge_index[1].astype(jnp.int32)
    pad = E_PAD - E
    src2 = jnp.concatenate([src, jnp.zeros((pad,), jnp.int32)]).reshape(EROWS, 128)
    dummy = DUMMY + (jnp.arange(pad, dtype=jnp.int32) % (NACC - DUMMY))
    dst2 = jnp.concatenate([dst, dummy]).reshape(EROWS, 128)

    deg2 = _deg_call()(dst2)
    deg_t = deg2.reshape(2, NACC).T
    hs3 = _mm1(deg_t, x, W1)
    agg = _agg_call()(hs3.reshape(2 * N, 128), src2, dst2)
    return _fin(deg_t, agg.reshape(2, N, 128), W2,
                b1.reshape(1, DH), b2.reshape(1, DO))


# speedup vs baseline: 1.0192x; 1.0068x over previous
import functools

import jax
import jax.numpy as jnp
from jax import lax
from jax.experimental import pallas as pl
from jax.experimental.pallas import tpu as pltpu
from jax.experimental.pallas import tpu_sc as plsc

N = 10000
E = 160000
DI = 256
DH = 256
DO = 128

E_PAD = 163840
EROWS = E_PAD // 128
NACC = 10240
DUMMY = 10008
MB = 5000
GRID_M = N // MB


def _deg_body(dst_hbm, out_hbm, dst_v, ones_v, zero_v, acc_sh, dsem):
    c = lax.axis_index("c")
    s = lax.axis_index("s")
    for i in range(8):
        ones_v[pl.ds(i * 16, 16)] = jnp.full((16,), 1.0, jnp.float32)

    def zfill(i, _):
        zero_v[pl.ds(i * 16, 16)] = jnp.zeros((16,), jnp.float32)
        return 0

    lax.fori_loop(0, 40, zfill, 0)
    pltpu.sync_copy(zero_v, acc_sh.at[pl.ds(pl.multiple_of(s * 640, 8), 640)])
    row0 = pl.multiple_of(c * 640 + s * 40, 8)
    pltpu.sync_copy(dst_hbm.at[pl.ds(row0, 40)], dst_v)
    plsc.subcore_barrier()

    def body(j, _):
        pltpu.async_copy(ones_v, acc_sh.at[dst_v.at[j]], dsem, add=True)
        return 0

    lax.fori_loop(0, 40, body, 0)

    def drain(j, _):
        pltpu.make_async_copy(ones_v, acc_sh.at[dst_v.at[j]], dsem).wait()
        return 0

    lax.fori_loop(0, 40, drain, 0)
    plsc.subcore_barrier()
    pltpu.sync_copy(acc_sh.at[pl.ds(pl.multiple_of(s * 640, 8), 640)],
                    out_hbm.at[pl.ds(pl.multiple_of(c * NACC + s * 640, 8), 640)])


@functools.cache
def _deg_call():
    mesh = plsc.VectorSubcoreMesh(core_axis_name="c", subcore_axis_name="s")
    return pl.kernel(
        _deg_body,
        mesh=mesh,
        out_type=jax.ShapeDtypeStruct((2 * NACC,), jnp.float32),
        scratch_types=[
            pltpu.VMEM((40, 128), jnp.int32),
            pltpu.VMEM((128,), jnp.float32),
            pltpu.VMEM((640,), jnp.float32),
            pltpu.VMEM_SHARED((NACC,), jnp.float32),
            pltpu.SemaphoreType.DMA,
        ],
    )


def _agg_body(hs_hbm, src_hbm, dst_hbm, out_hbm, src_v, dst_v, buf0_v, buf1_v,
              acc_sh, gs0, gs1, isem):
    c = lax.axis_index("c")
    s = lax.axis_index("s")
    r0 = pl.multiple_of(s * 624, 8)
    pltpu.async_copy(hs_hbm.at[pl.ds(pl.multiple_of(c * N + r0, 8), 624)],
                     acc_sh.at[pl.ds(r0, 624)], isem)

    @pl.when(s == 15)
    def _init_tail():
        pltpu.sync_copy(hs_hbm.at[pl.ds(pl.multiple_of(c * N + 9984, 8), 16)],
                        acc_sh.at[pl.ds(9984, 16)])

    off = c * N

    def load_idx(p):
        er = pl.multiple_of(s * 80 + p * 40, 8)
        pltpu.sync_copy(src_hbm.at[pl.ds(er, 40)], src_v)
        pltpu.sync_copy(dst_hbm.at[pl.ds(er, 40)], dst_v)

        def offs(r, _):
            for k in range(8):
                sl = pl.ds(k * 16, 16)
                src_v[r, sl] = src_v[r, sl] + off
            return 0

        lax.fori_loop(0, 40, offs, 0)

    load_idx(0)
    pltpu.async_copy(hs_hbm.at[src_v.at[0]], buf0_v, gs0)
    pltpu.make_async_copy(hs_hbm.at[pl.ds(pl.multiple_of(c * N + r0, 8), 624)],
                          acc_sh.at[pl.ds(r0, 624)], isem).wait()
    plsc.subcore_barrier()

    for p in range(2):
        if p:
            load_idx(p)
            pltpu.async_copy(hs_hbm.at[src_v.at[0]], buf0_v, gs0)

        def body(i, _):
            j0 = 2 * i
            j1 = 2 * i + 1
            pltpu.async_copy(hs_hbm.at[src_v.at[j1]], buf1_v, gs1)
            pltpu.make_async_copy(hs_hbm.at[src_v.at[j0]], buf0_v, gs0).wait()
            pltpu.sync_copy(buf0_v, acc_sh.at[dst_v.at[j0]], add=True)

            @pl.when(j1 + 1 < 40)
            def _prefetch():
                pltpu.async_copy(hs_hbm.at[src_v.at[j1 + 1]], buf0_v, gs0)

            pltpu.make_async_copy(hs_hbm.at[src_v.at[j1]], buf1_v, gs1).wait()
            pltpu.sync_copy(buf1_v, acc_sh.at[dst_v.at[j1]], add=True)
            return 0

        lax.fori_loop(0, 20, body, 0)
    plsc.subcore_barrier()
    pltpu.sync_copy(acc_sh.at[pl.ds(r0, 624)],
                    out_hbm.at[pl.ds(pl.multiple_of(c * N + r0, 8), 624)])

    @pl.when(s == 15)
    def _out_tail():
        pltpu.sync_copy(acc_sh.at[pl.ds(9984, 16)],
                        out_hbm.at[pl.ds(pl.multiple_of(c * N + 9984, 8), 16)])


@functools.cache
def _agg_call():
    mesh = plsc.VectorSubcoreMesh(core_axis_name="c", subcore_axis_name="s")
    return pl.kernel(
        _agg_body,
        mesh=mesh,
        out_type=jax.ShapeDtypeStruct((2 * N, DO), jnp.float32),
        scratch_types=[
            pltpu.VMEM((40, 128), jnp.int32),
            pltpu.VMEM((40, 128), jnp.int32),
            pltpu.VMEM((128, 128), jnp.float32),
            pltpu.VMEM((128, 128), jnp.float32),
            pltpu.VMEM_SHARED((NACC, 128), jnp.float32),
            pltpu.SemaphoreType.DMA,
            pltpu.SemaphoreType.DMA,
            pltpu.SemaphoreType.DMA,
        ],
    )


def _mm1_body(dt_ref, x_ref, w_ref, out_ref):
    d = dt_ref[:, 0:1] + dt_ref[:, 1:2] + 1.0
    dinv = lax.rsqrt(d)
    h = jnp.dot(x_ref[...], w_ref[...], preferred_element_type=jnp.float32)
    out_ref[0, :, :] = h * dinv


def _mm1(deg_t, x, W1):
    return pl.pallas_call(
        _mm1_body,
        grid=(GRID_M, 2),
        in_specs=[
            pl.BlockSpec((MB, 2), lambda i, j: (i, 0)),
            pl.BlockSpec((MB, DI), lambda i, j: (i, 0)),
            pl.BlockSpec((DI, 128), lambda i, j: (0, j)),
        ],
        out_specs=pl.BlockSpec((1, MB, 128), lambda i, j: (j, i, 0)),
        out_shape=jax.ShapeDtypeStruct((2, N, 128), jnp.float32),
    )(deg_t, x, W1)


def _fin_body(dt_ref, a_ref, w2_ref, b1_ref, b2_ref, out_ref):
    d = dt_ref[:, 0:1] + dt_ref[:, 1:2] + 1.0
    dinv = lax.rsqrt(d)
    zA = a_ref[0] * dinv + b1_ref[0:1, 0:128]
    zB = a_ref[1] * dinv + b1_ref[0:1, 128:256]
    zA = jnp.where(zA > 0, zA, jnp.exp(jnp.minimum(zA, 0.0)) - 1.0)
    zB = jnp.where(zB > 0, zB, jnp.exp(jnp.minimum(zB, 0.0)) - 1.0)
    y = (jnp.dot(zA, w2_ref[0:128, :], preferred_element_type=jnp.float32)
         + jnp.dot(zB, w2_ref[128:256, :], preferred_element_type=jnp.float32)
         + b2_ref[0:1, :])
    y = jnp.maximum(y, 0.0)
    m = jnp.max(y, axis=1, keepdims=True)
    e = y - m
    out_ref[...] = e - jnp.log(jnp.sum(jnp.exp(e), axis=1, keepdims=True))


def _fin(deg_t, agg, W2, b1, b2):
    return pl.pallas_call(
        _fin_body,
        grid=(GRID_M,),
        in_specs=[
            pl.BlockSpec((MB, 2), lambda i: (i, 0)),
            pl.BlockSpec((2, MB, 128), lambda i: (0, i, 0)),
            pl.BlockSpec((DH, DO), lambda i: (0, 0)),
            pl.BlockSpec((1, DH), lambda i: (0, 0)),
            pl.BlockSpec((1, DO), lambda i: (0, 0)),
        ],
        out_specs=pl.BlockSpec((MB, DO), lambda i: (i, 0)),
        out_shape=jax.ShapeDtypeStruct((N, DO), jnp.float32),
    )(deg_t, agg, W2, b1, b2)


def kernel(x, edge_index, W1, b1, W2, b2):
    src = edge_index[0].astype(jnp.int32)
    dst = edge_index[1].astype(jnp.int32)
    pad = E_PAD - E
    src2 = jnp.concatenate([src, jnp.zeros((pad,), jnp.int32)]).reshape(EROWS, 128)
    dummy = DUMMY + (jnp.arange(pad, dtype=jnp.int32) % (NACC - DUMMY))
    dst2 = jnp.concatenate([dst, dummy]).reshape(EROWS, 128)

    deg2 = _deg_call()(dst2)
    deg_t = deg2.reshape(2, NACC).T
    hs3 = _mm1(deg_t, x, W1)
    agg = _agg_call()(hs3.reshape(2 * N, 128), src2, dst2)
    return _fin(deg_t, agg.reshape(2, N, 128), W2,
                b1.reshape(1, DH), b2.reshape(1, DO))
